# baseline (speedup 1.0000x reference)
import functools
import jax
import jax.numpy as jnp
from jax import lax
from jax.experimental import pallas as pl
from jax.experimental.pallas import tpu as pltpu
from jax.experimental.pallas import tpu_sc as plsc

N = 100000
F = 128
G = 512
EPS = 1e-05
NC = 2
NS = 16
NW = NC * NS

R = 160
NBLK = N // R
JMAX = -(-NBLK // NW)
NCHUNK = R // 16
NSUB = 5
SUB = 32
NSTG = R + 16
NQ = NSTG // 16

_MESH = plsc.VectorSubcoreMesh(
    core_axis_name="c", subcore_axis_name="s", num_cores=NC, num_subcores=NS
)


def _sload(ref, i):
    return ref[pl.ds(i, 16)][0]


def _stage_ids(idx2d, idx1):
    idx1[pl.ds(0, 16)] = jnp.full((16,), -1, jnp.int32)
    for i in range(NCHUNK):
        idx1[pl.ds(16 + i * 16, 16)] = idx2d[i, :]
    idx1[pl.ds(16 + R, 16)] = jnp.full((16,), -2, jnp.int32)
    idx1[pl.ds(32 + R, 16)] = jnp.full((16,), -2, jnp.int32)
    idx1[pl.ds(48 + R, 16)] = jnp.full((16,), -2, jnp.int32)


def _count_runs(idx1):
    nruns = jnp.int32(0)
    for i in range(NCHUNK):
        ids_c = idx1[pl.ds(16 + i * 16, 16)]
        prv_c = idx1[pl.ds(15 + i * 16, 16)]
        m = ids_c != prv_c
        nruns = nruns + plsc.all_reduce_population_count(m)[0]
    return nruns


def _run_end(idx1, st, sid):
    def wbody(i, cc):
        w, found = cc
        iw = jnp.minimum(16 + w + 15, R + 63)
        has = jnp.where(_sload(idx1, iw) != sid, 1, 0)
        f2 = found | has
        w2 = jnp.where(f2 == 1, w, w + 16)
        return (w2, f2)

    w, _ = lax.fori_loop(0, NCHUNK + 1, wbody, (st, jnp.int32(0)))

    def sbody(i, cc):
        p, done = cc
        ip = jnp.minimum(16 + p, R + 63)
        neq = jnp.where(_sload(idx1, ip) != sid, 1, 0)
        done2 = done | neq
        return (jnp.where(done2 == 1, p, p + 1), done2)

    en, _ = lax.fori_loop(0, 16, sbody, (w, jnp.int32(0)))
    return jnp.minimum(en, R)


def _set_lane(stg2d, k, val):
    row = k >> 4
    lane = k & 15
    cur = stg2d[row, :]
    stg2d[row, :] = jnp.where(lax.iota(jnp.int32, 16) == lane, val, cur)


def _stats_sc(x_hbm, b_hbm, z128_hbm, z16_hbm,
              psum_hbm, psq_hbm, pcnt_hbm,
              idx2d, idx1, xb_v, stg_sum, stg_sq, stg_cnt, ids_stg,
              sp_sum, sp_sq, sp_cnt):
    c = lax.axis_index("c")
    s = lax.axis_index("s")
    wid = s * NC + c

    pltpu.sync_copy(z128_hbm.at[pl.ds(s * 32, 32)], sp_sum.at[pl.ds(s * 32, 32)])
    pltpu.sync_copy(z128_hbm.at[pl.ds(s * 32, 32)], sp_sq.at[pl.ds(s * 32, 32)])
    pltpu.sync_copy(z16_hbm.at[pl.ds(s * 32, 32)], sp_cnt.at[pl.ds(s * 32, 32)])
    for q in range(NQ):
        ids_stg[q, :] = jnp.full((16,), G, jnp.int32)
    plsc.subcore_barrier()

    for j in range(JMAX):
        blk = wid + NW * j

        @pl.when(blk < NBLK)
        def _():
            base = blk * R
            pltpu.sync_copy(b_hbm.at[blk], idx2d)
            pltpu.sync_copy(x_hbm.at[pl.ds(base, R)], xb_v)
            _stage_ids(idx2d, idx1)
            nruns = jnp.int32(5)

            def run_body(t, carry):
                st, k = carry
                sid = _sload(idx1, 16 + st)
                en = _run_end(idx1, st, sid)

                zero = jnp.zeros((16,), jnp.float32)
                for jj in range(F // 16):
                    stg_sum[k, pl.ds(jj * 16, 16)] = zero
                    stg_sq[k, pl.ds(jj * 16, 16)] = zero

                def acc_body(r, cc):
                    for jj in range(F // 16):
                        sl = pl.ds(jj * 16, 16)
                        v = xb_v[r, sl]
                        plsc.addupdate(stg_sum.at[k, sl], v)
                        plsc.addupdate(stg_sq.at[k, sl], v * v)
                    return cc

                lax.fori_loop(st, en, acc_body, 0)
                stg_cnt[k, :] = jnp.full((16,), 1.0, jnp.float32) * (
                    (en - st).astype(jnp.float32))
                _set_lane(ids_stg, k, jnp.clip(sid, 0, G - 1))
                return (en, k + 1)

            _, kf = lax.fori_loop(0, nruns, run_body,
                                  (jnp.int32(0), jnp.int32(0)))

            for q in range(NQ):
                @pl.when(q * 16 < kf)
                def _(q=q):
                    rows = pl.ds(q * 16, 16)
                    pltpu.sync_copy(stg_sum.at[rows], sp_sum.at[ids_stg.at[q]],
                                    add=True)
                    pltpu.sync_copy(stg_sq.at[rows], sp_sq.at[ids_stg.at[q]],
                                    add=True)
                    pltpu.sync_copy(stg_cnt.at[rows], sp_cnt.at[ids_stg.at[q]],
                                    add=True)
                    ids_stg[q, :] = jnp.full((16,), G, jnp.int32)

    plsc.subcore_barrier()
    pltpu.sync_copy(sp_sum.at[pl.ds(s * 32, 32)], psum_hbm.at[c, pl.ds(s * 32, 32)])
    pltpu.sync_copy(sp_sq.at[pl.ds(s * 32, 32)], psq_hbm.at[c, pl.ds(s * 32, 32)])
    pltpu.sync_copy(sp_cnt.at[pl.ds(s * 32, 32)], pcnt_hbm.at[c, pl.ds(s * 32, 32)])


_stats_call = functools.partial(
    pl.kernel,
    out_type=[
        jax.ShapeDtypeStruct((NC, G, F), jnp.float32),
        jax.ShapeDtypeStruct((NC, G, F), jnp.float32),
        jax.ShapeDtypeStruct((NC, G, 16), jnp.float32),
    ],
    mesh=_MESH,
    scratch_types=[
        pltpu.VMEM((NCHUNK, 16), jnp.int32),
        pltpu.VMEM((R + 64,), jnp.int32),
        pltpu.VMEM((R, F), jnp.float32),
        pltpu.VMEM((NSTG, F), jnp.float32),
        pltpu.VMEM((NSTG, F), jnp.float32),
        pltpu.VMEM((NSTG, 16), jnp.float32),
        pltpu.VMEM((NQ, 16), jnp.int32),
        pltpu.VMEM_SHARED((G + 1, F), jnp.float32),
        pltpu.VMEM_SHARED((G + 1, F), jnp.float32),
        pltpu.VMEM_SHARED((G + 1, 16), jnp.float32),
    ],
)(_stats_sc)


def _coef_tc(psum_ref, psq_ref, pcnt_ref, w_ref, b_ref, ms_ref, ab_ref):
    ssum = psum_ref[0] + psum_ref[1]
    ssq = psq_ref[0] + psq_ref[1]
    cnt = jnp.max(pcnt_ref[0] + pcnt_ref[1], axis=1, keepdims=True)
    c = jnp.maximum(cnt, 1.0)
    mean = ssum / c
    var = ssq / c - mean * mean
    rstd = lax.rsqrt(var + EPS)
    a = rstd * (ms_ref[...] * w_ref[...])
    bb = b_ref[...] - mean * a
    ab_ref[...] = jnp.concatenate([a, bb], axis=1)


def _apply_sc(x_hbm, b_hbm, a_hbm, bb_hbm, out_hbm,
              idx_v, xb_v, ca_v, cb_v, sp_a, sp_b, sem):
    c = lax.axis_index("c")
    s = lax.axis_index("s")
    wid = s * NC + c

    pltpu.sync_copy(a_hbm.at[pl.ds(s * 32, 32)], sp_a.at[pl.ds(s * 32, 32)])
    pltpu.sync_copy(bb_hbm.at[pl.ds(s * 32, 32)], sp_b.at[pl.ds(s * 32, 32)])
    plsc.subcore_barrier()

    for j in range(JMAX):
        blk = wid + NW * j

        @pl.when(blk < NBLK)
        def _():
            base = blk * R
            pltpu.sync_copy(b_hbm.at[blk], idx_v)
            descs = [
                pltpu.async_copy(
                    sp_a.at[idx_v.at[q]], ca_v.at[pl.ds(q * SUB, SUB)], sem
                )
                for q in range(NSUB)
            ] + [
                pltpu.async_copy(
                    sp_b.at[idx_v.at[q]], cb_v.at[pl.ds(q * SUB, SUB)], sem
                )
                for q in range(NSUB)
            ]
            pltpu.sync_copy(x_hbm.at[pl.ds(base, R)], xb_v)
            for d in descs:
                d.wait()

            def fma_row(r, carry):
                for jj in range(F // 16):
                    sl = pl.ds(jj * 16, 16)
                    xb_v[r, sl] = xb_v[r, sl] * ca_v[r, sl] + cb_v[r, sl]
                return carry

            lax.fori_loop(0, R, fma_row, 0)
            pltpu.sync_copy(xb_v, out_hbm.at[pl.ds(base, R)])


_apply_call = functools.partial(
    pl.kernel,
    out_type=jax.ShapeDtypeStruct((N, F), jnp.float32),
    mesh=_MESH,
    scratch_types=[
        pltpu.VMEM((NSUB, SUB), jnp.int32),
        pltpu.VMEM((R, F), jnp.float32),
        pltpu.VMEM((R, F), jnp.float32),
        pltpu.VMEM((R, F), jnp.float32),
        pltpu.VMEM_SHARED((G, F), jnp.float32),
        pltpu.VMEM_SHARED((G, F), jnp.float32),
        pltpu.SemaphoreType.DMA,
    ],
)(_apply_sc)


@jax.jit
def kernel(x, batch, weight, bias, mean_scale):
    bi = batch.astype(jnp.int32)
    b3 = bi.reshape(NBLK, NCHUNK, 16)
    b5 = bi.reshape(NBLK, NSUB, SUB)
    z128 = jnp.zeros((G, F), jnp.float32)
    z16 = jnp.zeros((G, 16), jnp.float32)

    psum, psq, pcnt = _stats_call(x, b3, z128, z16)

    ab = pl.pallas_call(
        _coef_tc,
        in_specs=[
            pl.BlockSpec((NC, G, F), lambda: (0, 0, 0)),
            pl.BlockSpec((NC, G, F), lambda: (0, 0, 0)),
            pl.BlockSpec((NC, G, 16), lambda: (0, 0, 0)),
            pl.BlockSpec((1, F), lambda: (0, 0)),
            pl.BlockSpec((1, F), lambda: (0, 0)),
            pl.BlockSpec((1, F), lambda: (0, 0)),
        ],
        out_specs=pl.BlockSpec((G, 2 * F), lambda: (0, 0)),
        out_shape=jax.ShapeDtypeStruct((G, 2 * F), jnp.float32),
    )(psum, psq, pcnt, weight.reshape(1, F), bias.reshape(1, F),
      mean_scale.reshape(1, F))

    return _apply_call(x, b5, ab[:, :F], ab[:, F:])


# speedup vs baseline: 1.4521x; 1.4521x over previous
import functools
import jax
import jax.numpy as jnp
from jax import lax
from jax.experimental import pallas as pl
from jax.experimental.pallas import tpu as pltpu
from jax.experimental.pallas import tpu_sc as plsc

N = 100000
F = 128
G = 512
EPS = 1e-05
NC = 2
NS = 16
NW = NC * NS

R = 160
NSUB = 5
SUB = 32
NBLK = N // R
JMAX = -(-NBLK // NW)

_MESH = plsc.VectorSubcoreMesh(
    core_axis_name="c", subcore_axis_name="s", num_cores=NC, num_subcores=NS
)


def _stats_sc(x_hbm, b_hbm, z128_hbm, z16_hbm, ones_hbm,
              psum_hbm, psq_hbm, pcnt_hbm,
              idx_v, xb_v, sq_v, ones_v, sp_sum, sp_sq, sp_cnt, sem):
    c = lax.axis_index("c")
    s = lax.axis_index("s")
    wid = s * NC + c

    pltpu.sync_copy(z128_hbm.at[pl.ds(s * 32, 32)], sp_sum.at[pl.ds(s * 32, 32)])
    pltpu.sync_copy(z128_hbm.at[pl.ds(s * 32, 32)], sp_sq.at[pl.ds(s * 32, 32)])
    pltpu.sync_copy(z16_hbm.at[pl.ds(s * 32, 32)], sp_cnt.at[pl.ds(s * 32, 32)])
    pltpu.sync_copy(ones_hbm, ones_v)
    plsc.subcore_barrier()

    for j in range(JMAX):
        blk = wid + NW * j

        @pl.when(blk < NBLK)
        def _():
            base = blk * R
            pltpu.sync_copy(b_hbm.at[blk], idx_v)
            pltpu.sync_copy(x_hbm.at[pl.ds(base, R)], xb_v)

            @plsc.parallel_loop(0, R, unroll=4)
            def _(r):
                for jj in range(F // 16):
                    v = xb_v[r, pl.ds(jj * 16, 16)]
                    sq_v[r, pl.ds(jj * 16, 16)] = v * v

            descs = []
            for q in range(NSUB):
                rows = pl.ds(q * SUB, SUB)
                descs.append(pltpu.async_copy(
                    xb_v.at[rows], sp_sum.at[idx_v.at[q]], sem, add=True))
                descs.append(pltpu.async_copy(
                    sq_v.at[rows], sp_sq.at[idx_v.at[q]], sem, add=True))
                descs.append(pltpu.async_copy(
                    ones_v.at[rows], sp_cnt.at[idx_v.at[q]], sem, add=True))
            for d in descs:
                d.wait()

    plsc.subcore_barrier()
    pltpu.sync_copy(sp_sum.at[pl.ds(s * 32, 32)], psum_hbm.at[c, pl.ds(s * 32, 32)])
    pltpu.sync_copy(sp_sq.at[pl.ds(s * 32, 32)], psq_hbm.at[c, pl.ds(s * 32, 32)])
    pltpu.sync_copy(sp_cnt.at[pl.ds(s * 32, 32)], pcnt_hbm.at[c, pl.ds(s * 32, 32)])


_stats_call = functools.partial(
    pl.kernel,
    out_type=[
        jax.ShapeDtypeStruct((NC, G, F), jnp.float32),
        jax.ShapeDtypeStruct((NC, G, F), jnp.float32),
        jax.ShapeDtypeStruct((NC, G, 16), jnp.float32),
    ],
    mesh=_MESH,
    scratch_types=[
        pltpu.VMEM((NSUB, SUB), jnp.int32),
        pltpu.VMEM((R, F), jnp.float32),
        pltpu.VMEM((R, F), jnp.float32),
        pltpu.VMEM((R, 16), jnp.float32),
        pltpu.VMEM_SHARED((G, F), jnp.float32),
        pltpu.VMEM_SHARED((G, F), jnp.float32),
        pltpu.VMEM_SHARED((G, 16), jnp.float32),
        pltpu.SemaphoreType.DMA,
    ],
)(_stats_sc)


def _coef_tc(psum_ref, psq_ref, pcnt_ref, w_ref, b_ref, ms_ref, ab_ref):
    ssum = psum_ref[0] + psum_ref[1]
    ssq = psq_ref[0] + psq_ref[1]
    cnt = jnp.max(pcnt_ref[0] + pcnt_ref[1], axis=1, keepdims=True)
    c = jnp.maximum(cnt, 1.0)
    mean = ssum / c
    var = ssq / c - mean * mean
    rstd = lax.rsqrt(var + EPS)
    a = rstd * (ms_ref[...] * w_ref[...])
    bb = b_ref[...] - mean * a
    ab_ref[...] = jnp.concatenate([a, bb], axis=1)


def _apply_sc(x_hbm, b_hbm, a_hbm, bb_hbm, out_hbm,
              idx_v, xb_v, ca_v, cb_v, sp_a, sp_b, sem):
    c = lax.axis_index("c")
    s = lax.axis_index("s")
    wid = s * NC + c

    pltpu.sync_copy(a_hbm.at[pl.ds(s * 32, 32)], sp_a.at[pl.ds(s * 32, 32)])
    pltpu.sync_copy(bb_hbm.at[pl.ds(s * 32, 32)], sp_b.at[pl.ds(s * 32, 32)])
    plsc.subcore_barrier()

    for j in range(JMAX):
        blk = wid + NW * j

        @pl.when(blk < NBLK)
        def _():
            base = blk * R
            pltpu.sync_copy(b_hbm.at[blk], idx_v)
            descs = [
                pltpu.async_copy(
                    sp_a.at[idx_v.at[q]], ca_v.at[pl.ds(q * SUB, SUB)], sem
                )
                for q in range(NSUB)
            ] + [
                pltpu.async_copy(
                    sp_b.at[idx_v.at[q]], cb_v.at[pl.ds(q * SUB, SUB)], sem
                )
                for q in range(NSUB)
            ]
            pltpu.sync_copy(x_hbm.at[pl.ds(base, R)], xb_v)
            for d in descs:
                d.wait()

            @plsc.parallel_loop(0, R, unroll=2)
            def _(r):
                for jj in range(F // 16):
                    sl = pl.ds(jj * 16, 16)
                    xb_v[r, sl] = xb_v[r, sl] * ca_v[r, sl] + cb_v[r, sl]
            pltpu.sync_copy(xb_v, out_hbm.at[pl.ds(base, R)])


_apply_call = functools.partial(
    pl.kernel,
    out_type=jax.ShapeDtypeStruct((N, F), jnp.float32),
    mesh=_MESH,
    scratch_types=[
        pltpu.VMEM((NSUB, SUB), jnp.int32),
        pltpu.VMEM((R, F), jnp.float32),
        pltpu.VMEM((R, F), jnp.float32),
        pltpu.VMEM((R, F), jnp.float32),
        pltpu.VMEM_SHARED((G, F), jnp.float32),
        pltpu.VMEM_SHARED((G, F), jnp.float32),
        pltpu.SemaphoreType.DMA,
    ],
)(_apply_sc)


@jax.jit
def kernel(x, batch, weight, bias, mean_scale):
    b3 = batch.astype(jnp.int32).reshape(NBLK, NSUB, SUB)
    z128 = jnp.zeros((G, F), jnp.float32)
    z16 = jnp.zeros((G, 16), jnp.float32)
    ones_h = jnp.ones((R, 16), jnp.float32)

    psum, psq, pcnt = _stats_call(x, b3, z128, z16, ones_h)

    ab = pl.pallas_call(
        _coef_tc,
        in_specs=[
            pl.BlockSpec((NC, G, F), lambda: (0, 0, 0)),
            pl.BlockSpec((NC, G, F), lambda: (0, 0, 0)),
            pl.BlockSpec((NC, G, 16), lambda: (0, 0, 0)),
            pl.BlockSpec((1, F), lambda: (0, 0)),
            pl.BlockSpec((1, F), lambda: (0, 0)),
            pl.BlockSpec((1, F), lambda: (0, 0)),
        ],
        out_specs=pl.BlockSpec((G, 2 * F), lambda: (0, 0)),
        out_shape=jax.ShapeDtypeStruct((G, 2 * F), jnp.float32),
    )(psum, psq, pcnt, weight.reshape(1, F), bias.reshape(1, F),
      mean_scale.reshape(1, F))

    return _apply_call(x, b3, ab[:, :F], ab[:, F:])


# speedup vs baseline: 1.6323x; 1.1241x over previous
import functools
import jax
import jax.numpy as jnp
from jax import lax
from jax.experimental import pallas as pl
from jax.experimental.pallas import tpu as pltpu
from jax.experimental.pallas import tpu_sc as plsc

N = 100000
F = 128
G = 512
EPS = 1e-05
NC = 2
NS = 16
NW = NC * NS

R = 160
NSUB = 5
SUB = 32
NBLK = N // R
JMAX = -(-NBLK // NW)

_MESH = plsc.VectorSubcoreMesh(
    core_axis_name="c", subcore_axis_name="s", num_cores=NC, num_subcores=NS
)


def _stats_sc(x_hbm, b_hbm, z128_hbm, z16_hbm, ones_hbm,
              psum_hbm, psq_hbm, pcnt_hbm,
              idx_v, xb_v, sq_v, ones_v, sp_sum, sp_sq, sp_cnt, sem):
    c = lax.axis_index("c")
    s = lax.axis_index("s")
    wid = s * NC + c

    pltpu.sync_copy(z128_hbm.at[pl.ds(s * 32, 32)], sp_sum.at[pl.ds(s * 32, 32)])
    pltpu.sync_copy(z128_hbm.at[pl.ds(s * 32, 32)], sp_sq.at[pl.ds(s * 32, 32)])
    pltpu.sync_copy(z16_hbm.at[pl.ds(s * 32, 32)], sp_cnt.at[pl.ds(s * 32, 32)])
    pltpu.sync_copy(ones_hbm, ones_v)
    plsc.subcore_barrier()

    for j in range(JMAX):
        blk = wid + NW * j

        @pl.when(blk < NBLK)
        def _():
            base = blk * R
            pltpu.sync_copy(b_hbm.at[blk], idx_v)
            pltpu.sync_copy(x_hbm.at[pl.ds(base, R)], xb_v)

            @plsc.parallel_loop(0, R, unroll=4)
            def _(r):
                for jj in range(F // 16):
                    v = xb_v[r, pl.ds(jj * 16, 16)]
                    sq_v[r, pl.ds(jj * 16, 16)] = v * v

            descs = []
            for q in range(NSUB):
                rows = pl.ds(q * SUB, SUB)
                descs.append(pltpu.async_copy(
                    xb_v.at[rows], sp_sum.at[idx_v.at[q]], sem, add=True))
                descs.append(pltpu.async_copy(
                    sq_v.at[rows], sp_sq.at[idx_v.at[q]], sem, add=True))
                descs.append(pltpu.async_copy(
                    ones_v.at[rows], sp_cnt.at[idx_v.at[q]], sem, add=True))
            for d in descs:
                d.wait()

    plsc.subcore_barrier()
    pltpu.sync_copy(sp_sum.at[pl.ds(s * 32, 32)], psum_hbm.at[c, pl.ds(s * 32, 32)])
    pltpu.sync_copy(sp_sq.at[pl.ds(s * 32, 32)], psq_hbm.at[c, pl.ds(s * 32, 32)])
    pltpu.sync_copy(sp_cnt.at[pl.ds(s * 32, 32)], pcnt_hbm.at[c, pl.ds(s * 32, 32)])


_stats_call = functools.partial(
    pl.kernel,
    out_type=[
        jax.ShapeDtypeStruct((NC, G, F), jnp.float32),
        jax.ShapeDtypeStruct((NC, G, F), jnp.float32),
        jax.ShapeDtypeStruct((NC, G, 16), jnp.float32),
    ],
    mesh=_MESH,
    scratch_types=[
        pltpu.VMEM((NSUB, SUB), jnp.int32),
        pltpu.VMEM((R, F), jnp.float32),
        pltpu.VMEM((R, F), jnp.float32),
        pltpu.VMEM((R, 16), jnp.float32),
        pltpu.VMEM_SHARED((G, F), jnp.float32),
        pltpu.VMEM_SHARED((G, F), jnp.float32),
        pltpu.VMEM_SHARED((G, 16), jnp.float32),
        pltpu.SemaphoreType.DMA,
    ],
)(_stats_sc)


def _coef_tc(psum_ref, psq_ref, pcnt_ref, w_ref, b_ref, ms_ref, ab_ref):
    ssum = psum_ref[0] + psum_ref[1]
    ssq = psq_ref[0] + psq_ref[1]
    cnt = jnp.max(pcnt_ref[0] + pcnt_ref[1], axis=1, keepdims=True)
    c = jnp.maximum(cnt, 1.0)
    mean = ssum / c
    var = ssq / c - mean * mean
    rstd = lax.rsqrt(var + EPS)
    a = rstd * (ms_ref[...] * w_ref[...])
    bb = b_ref[...] - mean * a
    ab_ref[...] = jnp.concatenate([a, bb], axis=1)


RC = 160
NSUBC = 5
SUBC = 32
NBLKC = N // RC
NMAIN = NBLKC // NW
NTAIL = NBLKC - NMAIN * NW


def _apply_sc(x_hbm, b_hbm, a_hbm, bb_hbm, out_hbm,
              idxa, idxb, xb2, ca_v, cb_v, sp_a, sp_b, semx, semi, semg, semo):
    c = lax.axis_index("c")
    s = lax.axis_index("s")
    wid = s * NC + c

    pltpu.sync_copy(a_hbm.at[pl.ds(s * 32, 32)], sp_a.at[pl.ds(s * 32, 32)])
    pltpu.sync_copy(bb_hbm.at[pl.ds(s * 32, 32)], sp_b.at[pl.ds(s * 32, 32)])
    plsc.subcore_barrier()

    idxbuf = [idxa, idxb]

    def fire_gathers(p):
        ds_ = []
        for q in range(NSUBC):
            rows = pl.ds(q * SUBC, SUBC)
            ds_.append(pltpu.async_copy(
                sp_a.at[idxbuf[p].at[q]], ca_v.at[rows], semg))
            ds_.append(pltpu.async_copy(
                sp_b.at[idxbuf[p].at[q]], cb_v.at[rows], semg))
        return ds_

    def fma(p):
        @plsc.parallel_loop(0, RC, unroll=1)
        def _(r):
            for jj in range(F // 16):
                sl = pl.ds(jj * 16, 16)
                xb2[p, r, sl] = xb2[p, r, sl] * ca_v[r, sl] + cb_v[r, sl]

    pltpu.sync_copy(b_hbm.at[wid], idxa)
    dg = fire_gathers(0)
    dx = [None, None]
    didx = [None, None]
    dout = [None, None]
    dx[0] = pltpu.async_copy(x_hbm.at[pl.ds(wid * RC, RC)], xb2.at[0], semx)

    for j in range(NMAIN):
        p = j & 1
        blk = wid + NW * j
        if j + 1 < NMAIN:
            if dout[1 - p] is not None:
                dout[1 - p].wait()
            nblk = wid + NW * (j + 1)
            didx[1 - p] = pltpu.async_copy(b_hbm.at[nblk], idxbuf[1 - p], semi)
            dx[1 - p] = pltpu.async_copy(
                x_hbm.at[pl.ds(nblk * RC, RC)], xb2.at[1 - p], semx)
        for d in dg:
            d.wait()
        dx[p].wait()
        fma(p)
        dout[p] = pltpu.async_copy(
            xb2.at[p], out_hbm.at[pl.ds(blk * RC, RC)], semo)
        if j + 1 < NMAIN:
            didx[1 - p].wait()
            dg = fire_gathers(1 - p)
    for d in dout:
        if d is not None:
            d.wait()

    @pl.when(wid < NTAIL)
    def _():
        blk = NMAIN * NW + wid
        base = blk * RC
        pltpu.sync_copy(b_hbm.at[blk], idxa)
        pltpu.sync_copy(x_hbm.at[pl.ds(base, RC)], xb2.at[0])
        tds = fire_gathers(0)
        for d in tds:
            d.wait()
        fma(0)
        pltpu.sync_copy(xb2.at[0], out_hbm.at[pl.ds(base, RC)])


_apply_call = functools.partial(
    pl.kernel,
    out_type=jax.ShapeDtypeStruct((N, F), jnp.float32),
    mesh=_MESH,
    scratch_types=[
        pltpu.VMEM((NSUBC, SUBC), jnp.int32),
        pltpu.VMEM((NSUBC, SUBC), jnp.int32),
        pltpu.VMEM((2, RC, F), jnp.float32),
        pltpu.VMEM((RC, F), jnp.float32),
        pltpu.VMEM((RC, F), jnp.float32),
        pltpu.VMEM_SHARED((G, F), jnp.float32),
        pltpu.VMEM_SHARED((G, F), jnp.float32),
        pltpu.SemaphoreType.DMA,
        pltpu.SemaphoreType.DMA,
        pltpu.SemaphoreType.DMA,
        pltpu.SemaphoreType.DMA,
    ],
)(_apply_sc)


@jax.jit
def kernel(x, batch, weight, bias, mean_scale):
    bi = batch.astype(jnp.int32)
    b3 = bi.reshape(NBLK, NSUB, SUB)
    b5 = bi.reshape(NBLKC, NSUBC, SUBC)
    z128 = jnp.zeros((G, F), jnp.float32)
    z16 = jnp.zeros((G, 16), jnp.float32)
    ones_h = jnp.ones((R, 16), jnp.float32)

    psum, psq, pcnt = _stats_call(x, b3, z128, z16, ones_h)

    ab = pl.pallas_call(
        _coef_tc,
        in_specs=[
            pl.BlockSpec((NC, G, F), lambda: (0, 0, 0)),
            pl.BlockSpec((NC, G, F), lambda: (0, 0, 0)),
            pl.BlockSpec((NC, G, 16), lambda: (0, 0, 0)),
            pl.BlockSpec((1, F), lambda: (0, 0)),
            pl.BlockSpec((1, F), lambda: (0, 0)),
            pl.BlockSpec((1, F), lambda: (0, 0)),
        ],
        out_specs=pl.BlockSpec((G, 2 * F), lambda: (0, 0)),
        out_shape=jax.ShapeDtypeStruct((G, 2 * F), jnp.float32),
    )(psum, psq, pcnt, weight.reshape(1, F), bias.reshape(1, F),
      mean_scale.reshape(1, F))

    return _apply_call(x, b5, ab[:, :F], ab[:, F:])


# speedup vs baseline: 1.8084x; 1.1078x over previous
import functools
import jax
import jax.numpy as jnp
from jax import lax
from jax.experimental import pallas as pl
from jax.experimental.pallas import tpu as pltpu
from jax.experimental.pallas import tpu_sc as plsc

N = 100000
F = 128
G = 512
EPS = 1e-05
NC = 2
NS = 16
NW = NC * NS

R = 160
NSUB = 5
SUB = 32
NBLK = N // R
JMAX = -(-NBLK // NW)

_MESH = plsc.VectorSubcoreMesh(
    core_axis_name="c", subcore_axis_name="s", num_cores=NC, num_subcores=NS
)


NMAINS = NBLK // NW
NTAILS = NBLK - NMAINS * NW


def _stats_sc(x_hbm, b_hbm, z128_hbm, z16_hbm, ones_hbm,
              psum_hbm, psq_hbm, pcnt_hbm,
              idxa, idxb, xb2, sq_v, ones_v, sp_sum, sp_sq, sp_cnt,
              semx, semi, sems):
    c = lax.axis_index("c")
    s = lax.axis_index("s")
    wid = s * NC + c
    idxbuf = [idxa, idxb]

    pltpu.sync_copy(z128_hbm.at[pl.ds(s * 32, 32)], sp_sum.at[pl.ds(s * 32, 32)])
    pltpu.sync_copy(z128_hbm.at[pl.ds(s * 32, 32)], sp_sq.at[pl.ds(s * 32, 32)])
    pltpu.sync_copy(z16_hbm.at[pl.ds(s * 32, 32)], sp_cnt.at[pl.ds(s * 32, 32)])
    pltpu.sync_copy(ones_hbm, ones_v)
    plsc.subcore_barrier()

    def compute_sq(p):
        @plsc.parallel_loop(0, R, unroll=2)
        def _(r):
            for jj in range(F // 16):
                v = xb2[p, r, pl.ds(jj * 16, 16)]
                sq_v[r, pl.ds(jj * 16, 16)] = v * v

    def fire_scatters(p):
        ds_ = []
        for q in range(NSUB):
            rows = pl.ds(q * SUB, SUB)
            ds_.append(pltpu.async_copy(
                xb2.at[p].at[rows], sp_sum.at[idxbuf[p].at[q]], sems, add=True))
            ds_.append(pltpu.async_copy(
                sq_v.at[rows], sp_sq.at[idxbuf[p].at[q]], sems, add=True))
            ds_.append(pltpu.async_copy(
                ones_v.at[rows], sp_cnt.at[idxbuf[p].at[q]], sems, add=True))
        return ds_

    pltpu.sync_copy(b_hbm.at[wid], idxa)
    dx = [None, None]
    didx = [None, None]
    dx[0] = pltpu.async_copy(x_hbm.at[pl.ds(wid * R, R)], xb2.at[0], semx)
    dsc = []

    for j in range(NMAINS):
        p = j & 1
        dx[p].wait()
        for d in dsc:
            d.wait()
        if j + 1 < NMAINS:
            nblk = wid + NW * (j + 1)
            didx[1 - p] = pltpu.async_copy(b_hbm.at[nblk], idxbuf[1 - p], semi)
            dx[1 - p] = pltpu.async_copy(
                x_hbm.at[pl.ds(nblk * R, R)], xb2.at[1 - p], semx)
        compute_sq(p)
        if j > 0:
            didx[p].wait()
        dsc = fire_scatters(p)
    for d in dsc:
        d.wait()

    @pl.when(wid < NTAILS)
    def _():
        blk = NMAINS * NW + wid
        base = blk * R
        pltpu.sync_copy(b_hbm.at[blk], idxa)
        pltpu.sync_copy(x_hbm.at[pl.ds(base, R)], xb2.at[0])
        compute_sq(0)
        tds = fire_scatters(0)
        for d in tds:
            d.wait()

    plsc.subcore_barrier()
    pltpu.sync_copy(sp_sum.at[pl.ds(s * 32, 32)], psum_hbm.at[c, pl.ds(s * 32, 32)])
    pltpu.sync_copy(sp_sq.at[pl.ds(s * 32, 32)], psq_hbm.at[c, pl.ds(s * 32, 32)])
    pltpu.sync_copy(sp_cnt.at[pl.ds(s * 32, 32)], pcnt_hbm.at[c, pl.ds(s * 32, 32)])


_stats_call = functools.partial(
    pl.kernel,
    out_type=[
        jax.ShapeDtypeStruct((NC, G, F), jnp.float32),
        jax.ShapeDtypeStruct((NC, G, F), jnp.float32),
        jax.ShapeDtypeStruct((NC, G, 16), jnp.float32),
    ],
    mesh=_MESH,
    scratch_types=[
        pltpu.VMEM((NSUB, SUB), jnp.int32),
        pltpu.VMEM((NSUB, SUB), jnp.int32),
        pltpu.VMEM((2, R, F), jnp.float32),
        pltpu.VMEM((R, F), jnp.float32),
        pltpu.VMEM((R, 16), jnp.float32),
        pltpu.VMEM_SHARED((G, F), jnp.float32),
        pltpu.VMEM_SHARED((G, F), jnp.float32),
        pltpu.VMEM_SHARED((G, 16), jnp.float32),
        pltpu.SemaphoreType.DMA,
        pltpu.SemaphoreType.DMA,
        pltpu.SemaphoreType.DMA,
    ],
)(_stats_sc)


def _coef_tc(psum_ref, psq_ref, pcnt_ref, w_ref, b_ref, ms_ref, ab_ref):
    ssum = psum_ref[0] + psum_ref[1]
    ssq = psq_ref[0] + psq_ref[1]
    cnt = jnp.max(pcnt_ref[0] + pcnt_ref[1], axis=1, keepdims=True)
    c = jnp.maximum(cnt, 1.0)
    mean = ssum / c
    var = ssq / c - mean * mean
    rstd = lax.rsqrt(var + EPS)
    a = rstd * (ms_ref[...] * w_ref[...])
    bb = b_ref[...] - mean * a
    ab_ref[...] = jnp.concatenate([a, bb], axis=1)


RC = 160
NSUBC = 5
SUBC = 32
NBLKC = N // RC
NMAIN = NBLKC // NW
NTAIL = NBLKC - NMAIN * NW


def _apply_sc(x_hbm, b_hbm, a_hbm, bb_hbm, out_hbm,
              idxa, idxb, xb2, ca_v, cb_v, sp_a, sp_b, semx, semi, semg, semo):
    c = lax.axis_index("c")
    s = lax.axis_index("s")
    wid = s * NC + c

    pltpu.sync_copy(a_hbm.at[pl.ds(s * 32, 32)], sp_a.at[pl.ds(s * 32, 32)])
    pltpu.sync_copy(bb_hbm.at[pl.ds(s * 32, 32)], sp_b.at[pl.ds(s * 32, 32)])
    plsc.subcore_barrier()

    idxbuf = [idxa, idxb]

    def fire_gathers(p):
        ds_ = []
        for q in range(NSUBC):
            rows = pl.ds(q * SUBC, SUBC)
            ds_.append(pltpu.async_copy(
                sp_a.at[idxbuf[p].at[q]], ca_v.at[rows], semg))
            ds_.append(pltpu.async_copy(
                sp_b.at[idxbuf[p].at[q]], cb_v.at[rows], semg))
        return ds_

    def fma(p):
        @plsc.parallel_loop(0, RC, unroll=1)
        def _(r):
            for jj in range(F // 16):
                sl = pl.ds(jj * 16, 16)
                xb2[p, r, sl] = xb2[p, r, sl] * ca_v[r, sl] + cb_v[r, sl]

    pltpu.sync_copy(b_hbm.at[wid], idxa)
    dg = fire_gathers(0)
    dx = [None, None]
    didx = [None, None]
    dout = [None, None]
    dx[0] = pltpu.async_copy(x_hbm.at[pl.ds(wid * RC, RC)], xb2.at[0], semx)

    for j in range(NMAIN):
        p = j & 1
        blk = wid + NW * j
        if j + 1 < NMAIN:
            if dout[1 - p] is not None:
                dout[1 - p].wait()
            nblk = wid + NW * (j + 1)
            didx[1 - p] = pltpu.async_copy(b_hbm.at[nblk], idxbuf[1 - p], semi)
            dx[1 - p] = pltpu.async_copy(
                x_hbm.at[pl.ds(nblk * RC, RC)], xb2.at[1 - p], semx)
        for d in dg:
            d.wait()
        dx[p].wait()
        fma(p)
        dout[p] = pltpu.async_copy(
            xb2.at[p], out_hbm.at[pl.ds(blk * RC, RC)], semo)
        if j + 1 < NMAIN:
            didx[1 - p].wait()
            dg = fire_gathers(1 - p)
    for d in dout:
        if d is not None:
            d.wait()

    @pl.when(wid < NTAIL)
    def _():
        blk = NMAIN * NW + wid
        base = blk * RC
        pltpu.sync_copy(b_hbm.at[blk], idxa)
        pltpu.sync_copy(x_hbm.at[pl.ds(base, RC)], xb2.at[0])
        tds = fire_gathers(0)
        for d in tds:
            d.wait()
        fma(0)
        pltpu.sync_copy(xb2.at[0], out_hbm.at[pl.ds(base, RC)])


_apply_call = functools.partial(
    pl.kernel,
    out_type=jax.ShapeDtypeStruct((N, F), jnp.float32),
    mesh=_MESH,
    scratch_types=[
        pltpu.VMEM((NSUBC, SUBC), jnp.int32),
        pltpu.VMEM((NSUBC, SUBC), jnp.int32),
        pltpu.VMEM((2, RC, F), jnp.float32),
        pltpu.VMEM((RC, F), jnp.float32),
        pltpu.VMEM((RC, F), jnp.float32),
        pltpu.VMEM_SHARED((G, F), jnp.float32),
        pltpu.VMEM_SHARED((G, F), jnp.float32),
        pltpu.SemaphoreType.DMA,
        pltpu.SemaphoreType.DMA,
        pltpu.SemaphoreType.DMA,
        pltpu.SemaphoreType.DMA,
    ],
)(_apply_sc)


@jax.jit
def kernel(x, batch, weight, bias, mean_scale):
    bi = batch.astype(jnp.int32)
    b3 = bi.reshape(NBLK, NSUB, SUB)
    b5 = bi.reshape(NBLKC, NSUBC, SUBC)
    z128 = jnp.zeros((G, F), jnp.float32)
    z16 = jnp.zeros((G, 16), jnp.float32)
    ones_h = jnp.ones((R, 16), jnp.float32)

    psum, psq, pcnt = _stats_call(x, b3, z128, z16, ones_h)

    ab = pl.pallas_call(
        _coef_tc,
        in_specs=[
            pl.BlockSpec((NC, G, F), lambda: (0, 0, 0)),
            pl.BlockSpec((NC, G, F), lambda: (0, 0, 0)),
            pl.BlockSpec((NC, G, 16), lambda: (0, 0, 0)),
            pl.BlockSpec((1, F), lambda: (0, 0)),
            pl.BlockSpec((1, F), lambda: (0, 0)),
            pl.BlockSpec((1, F), lambda: (0, 0)),
        ],
        out_specs=pl.BlockSpec((G, 2 * F), lambda: (0, 0)),
        out_shape=jax.ShapeDtypeStruct((G, 2 * F), jnp.float32),
    )(psum, psq, pcnt, weight.reshape(1, F), bias.reshape(1, F),
      mean_scale.reshape(1, F))

    return _apply_call(x, b5, ab[:, :F], ab[:, F:])
